# baseline (speedup 1.0000x reference)
import functools

import jax
import jax.numpy as jnp
from jax import lax
from jax.experimental import pallas as pl
from jax.experimental.pallas import tpu as pltpu
from jax.experimental.pallas import tpu_sc as plsc

D = 64
ROW_LEN = 50
ROW_PAD = 56
NUM_WORKERS = 32
CW = 128
L = 16


def _make_sc_kernel(num_cols):
    cols_per_w = num_cols // NUM_WORKERS
    n_cc = cols_per_w // CW
    total = ROW_LEN * n_cc
    mesh = plsc.VectorSubcoreMesh(core_axis_name="c", subcore_axis_name="s")

    @functools.partial(
        pl.kernel,
        out_type=jax.ShapeDtypeStruct((ROW_LEN, D, num_cols), jnp.float32),
        mesh=mesh,
        scratch_types=[
            pltpu.VMEM((ROW_PAD, cols_per_w), jnp.int32),
            pltpu.VMEM((2, CW, D), jnp.float32),
            pltpu.VMEM((2, D, CW), jnp.float32),
            pltpu.SemaphoreType.DMA((2,)),
            pltpu.SemaphoreType.DMA((2,)),
        ],
        compiler_params=pltpu.CompilerParams(
            use_tc_tiling_on_sc=False, needs_layout_passes=False),
    )
    def emb(table_hbm, idx_hbm, out_hbm, idx_v, rows_v, tbuf, gsem, osem):
        wid = lax.axis_index("s") * 2 + lax.axis_index("c")
        col0 = wid * cols_per_w

        pltpu.sync_copy(idx_hbm.at[:, pl.ds(col0, cols_per_w)], idx_v)

        def g_src(u):
            t = u // n_cc
            cc = u % n_cc
            return table_hbm.at[idx_v.at[t, pl.ds(cc * CW, CW)]]

        def g_start(u, b):
            pltpu.async_copy(g_src(u), rows_v.at[b], gsem.at[b])

        def g_wait(u, b):
            pltpu.make_async_copy(g_src(u), rows_v.at[b], gsem.at[b]).wait()

        def o_dst(u):
            t = u // n_cc
            cc = u % n_cc
            return out_hbm.at[t, :, pl.ds(col0 + cc * CW, CW)]

        def o_start(u, b):
            pltpu.async_copy(tbuf.at[b], o_dst(u), osem.at[b])

        def o_wait(u, b):
            pltpu.make_async_copy(tbuf.at[b], o_dst(u), osem.at[b]).wait()

        row_ids = [jnp.arange(L, dtype=jnp.int32) + q * L for q in range(CW // L)]

        g_start(0, 0)

        @pl.loop(0, total)
        def unit(u):
            b = u % 2

            @pl.when(u + 1 < total)
            def _():
                g_start(u + 1, 1 - b)

            g_wait(u, b)

            @pl.when(u >= 2)
            def _():
                o_wait(u - 2, b)

            for d in range(D):
                dvec = jnp.full((L,), d, dtype=jnp.int32)
                for q in range(CW // L):
                    v = plsc.load_gather(rows_v.at[b], [row_ids[q], dvec])
                    tbuf[b, d, pl.ds(q * L, L)] = v

            o_start(u, b)

        o_wait(total - 2, 0)
        o_wait(total - 1, 1)

    return emb


def kernel(x, W_embed):
    b0, b1 = x.shape
    xp = jnp.pad(x.astype(jnp.int32).T, ((0, ROW_PAD - b1), (0, 0)))
    out_t = _make_sc_kernel(b0)(W_embed, xp)
    return jnp.transpose(out_t, (2, 0, 1))


# speedup vs baseline: 1.6062x; 1.6062x over previous
import functools

import jax
import jax.numpy as jnp
from jax import lax
from jax.experimental import pallas as pl
from jax.experimental.pallas import tpu as pltpu
from jax.experimental.pallas import tpu_sc as plsc

D = 64
ROW_LEN = 50
IDX_PAD = 64
GATHER_W = 56
NUM_WORKERS = 32
K = 8


def _idx_prep(x):
    n = x.shape[0]
    br = 2048

    def body(x_ref, o_ref):
        xb = x_ref[...]
        o_ref[...] = jnp.concatenate(
            [xb, xb[:, ROW_LEN - (IDX_PAD - ROW_LEN):]], axis=1)

    return pl.pallas_call(
        body,
        grid=(n // br,),
        in_specs=[pl.BlockSpec((br, ROW_LEN), lambda i: (i, 0))],
        out_specs=pl.BlockSpec((br, IDX_PAD), lambda i: (i, 0)),
        out_shape=jax.ShapeDtypeStruct((n, IDX_PAD), jnp.int32),
    )(x)


def _make_sc_kernel(num_rows):
    rows_per_w = num_rows // NUM_WORKERS
    num_chunks = rows_per_w // K
    assert rows_per_w % K == 0 and num_chunks % 2 == 0 and num_chunks >= 4
    mesh = plsc.VectorSubcoreMesh(core_axis_name="c", subcore_axis_name="s")

    @functools.partial(
        pl.kernel,
        out_type=jax.ShapeDtypeStruct((num_rows, ROW_LEN, D), jnp.float32),
        mesh=mesh,
        scratch_types=[
            pltpu.VMEM((2, K, IDX_PAD), jnp.int32),
            pltpu.VMEM((2, K, GATHER_W, D), jnp.float32),
            pltpu.SemaphoreType.DMA,
            pltpu.SemaphoreType.DMA((2,)),
            pltpu.SemaphoreType.DMA((2,)),
        ],
        compiler_params=pltpu.CompilerParams(use_tc_tiling_on_sc=False),
    )
    def emb(table_hbm, idx_hbm, out_hbm, idx_v, rows_v, gsem, isem, osem):
        wid = lax.axis_index("s") * 2 + lax.axis_index("c")
        base_row = wid * rows_per_w

        def idx_start(c, b):
            pltpu.async_copy(
                idx_hbm.at[pl.ds(base_row + c * K, K)],
                idx_v.at[b], isem.at[b])

        def idx_wait(c, b):
            pltpu.make_async_copy(
                idx_hbm.at[pl.ds(base_row + c * K, K)],
                idx_v.at[b], isem.at[b]).wait()

        def out_start(c, b):
            pltpu.async_copy(
                rows_v.at[b, :, pl.ds(0, ROW_LEN)],
                out_hbm.at[pl.ds(base_row + c * K, K)], osem.at[b])

        def out_wait(c, b):
            pltpu.make_async_copy(
                rows_v.at[b, :, pl.ds(0, ROW_LEN)],
                out_hbm.at[pl.ds(base_row + c * K, K)], osem.at[b]).wait()

        idx_start(0, 0)
        idx_start(1, 1)

        @pl.loop(0, num_chunks)
        def chunk(c):
            b = c % 2
            idx_wait(c, b)

            @pl.when(c >= 2)
            def _():
                out_wait(c - 2, b)

            @pl.loop(0, K)
            def fire(j):
                pltpu.async_copy(
                    table_hbm.at[idx_v.at[b, j, pl.ds(0, GATHER_W)]],
                    rows_v.at[b, j], gsem)

            @pl.loop(0, K)
            def drain(j):
                pltpu.make_async_copy(
                    table_hbm.at[idx_v.at[b, j, pl.ds(0, GATHER_W)]],
                    rows_v.at[b, j], gsem).wait()

            out_start(c, b)

            @pl.when(c + 2 < num_chunks)
            def _():
                idx_start(c + 2, b)

        out_wait(num_chunks - 2, 0)
        out_wait(num_chunks - 1, 1)

    return emb


def kernel(x, W_embed):
    b0, _ = x.shape
    idx = _idx_prep(x.astype(jnp.int32))
    return _make_sc_kernel(b0)(W_embed, idx)
